# baseline (speedup 1.0000x reference)
import jax
import jax.numpy as jnp
from jax.experimental import pallas as pl
from jax.experimental.pallas import tpu as pltpu

_B, _S, _D_IN, _HID, _HEADS = 16, 512, 256, 256, 4
_DH = _HID // _HEADS
_NEG = -1e9
_L = 1.4426950408889634
_NEGL = _NEG * _L
_BUCKETS = (64, 128, 192, 256, 320, 384, 448, 512)


def _for_bucket(cnt, fn):
    lo = 0
    for iw in _BUCKETS:
        cond = (cnt <= iw) if lo == 0 else ((cnt > lo) & (cnt <= iw))
        pl.when(cond)(lambda iw=iw: fn(iw))
        lo = iw


def _body(x_ref, adj_ref, counts_ref, W1_ref, As1_ref, Ad1_ref, b1_ref,
          W2_ref, As2_ref, Ad2_ref, b2_ref, out_ref, abuf_ref, sem):
    b = pl.program_id(0)
    n = counts_ref[b]
    slot = jax.lax.rem(b, 2)

    def corner_copy(nb, sl, iw):
        cw = -(-iw // 128) * 128
        return pltpu.make_async_copy(
            adj_ref.at[nb, pl.ds(0, iw), pl.ds(0, cw)],
            abuf_ref.at[sl, pl.ds(0, iw), pl.ds(0, cw)],
            sem.at[sl])

    def start_for(nb, sl):
        _for_bucket(counts_ref[nb], lambda iw: corner_copy(nb, sl, iw).start())

    pl.when(b == 0)(lambda: start_for(0, 0))
    pl.when(b + 1 < _B)(lambda: start_for(b + 1, jax.lax.rem(b + 1, 2)))

    def arm(iw):
        corner_copy(b, slot, iw).wait()
        adq = abuf_ref[slot, :iw, :iw].astype(jnp.float32)
        adqt = adq.T
        jj = jax.lax.broadcasted_iota(jnp.int32, (iw, iw), 0)
        ii = jax.lax.broadcasted_iota(jnp.int32, (iw, iw), 1)
        mb = jnp.where((adqt > 0.5) | (ii == jj), 0.0, _NEGL)
        icol = jax.lax.broadcasted_iota(jnp.int32, (1, iw), 1)

        def gat(xin, W_ref, As_ref, Ad_ref, b_ref):
            h = jnp.dot(xin, W_ref[...], preferred_element_type=jnp.float32)
            al_s = jnp.dot(h, As_ref[...], preferred_element_type=jnp.float32)
            al_d = jnp.dot(h, Ad_ref[...], preferred_element_type=jnp.float32)
            al_sr = jnp.where(icol < n, al_s.T, _NEG)
            as1 = _L * al_sr
            as2 = (0.2 * _L) * al_sr
            amax = jnp.max(al_sr, axis=1, keepdims=True)
            outs = []
            for hd in range(_HEADS):
                ad = al_d[:, hd:hd + 1]
                m1 = ad + amax[hd:hd + 1, :]
                mx = jnp.maximum(m1, 0.2 * m1)
                ad1 = _L * (ad - mx)
                ad2 = _L * (0.2 * ad - mx)
                t = jnp.maximum(as1[hd:hd + 1, :] + ad1,
                                as2[hd:hd + 1, :] + ad2) + mb
                ex = jnp.exp2(t)
                den = jnp.sum(ex, axis=1, keepdims=True)
                rden = 1.0 / jnp.maximum(den, 1e-30)
                outs.append(jnp.dot(ex, h[:, hd * _DH:(hd + 1) * _DH],
                                    preferred_element_type=jnp.float32) * rden)
            return jnp.concatenate(outs, axis=1) + b_ref[...]

        x1 = gat(x_ref[0, :iw, :], W1_ref, As1_ref, Ad1_ref, b1_ref)
        x2 = gat(x1, W2_ref, As2_ref, Ad2_ref, b2_ref)
        rows = jax.lax.broadcasted_iota(jnp.int32, (iw, 1), 0)
        out_ref[0, :iw, :] = x2 * (rows < n).astype(jnp.float32)
        if iw < _S:
            out_ref[0, iw:, :] = jnp.zeros((_S - iw, _HID), jnp.float32)

    _for_bucket(n, arm)


def _head_mat(a):
    k = jnp.arange(_HID)
    sel = (k[:, None] // _DH) == jnp.arange(_HEADS)[None, :]
    return a.reshape(_HID)[:, None] * sel.astype(a.dtype)


def kernel(sent_emb, adj_mask, sent_counts, W1, a1_src, a1_dst, b1,
           W2, a2_src, a2_dst, b2):
    As1, Ad1 = _head_mat(a1_src), _head_mat(a1_dst)
    As2, Ad2 = _head_mat(a2_src), _head_mat(a2_dst)
    full = lambda shape: pl.BlockSpec(shape, lambda b: (0,) * len(shape))
    out = pl.pallas_call(
        _body,
        grid=(_B,),
        in_specs=[
            pl.BlockSpec((1, _S, _D_IN), lambda b: (b, 0, 0)),
            pl.BlockSpec(memory_space=pl.ANY),
            pl.BlockSpec(memory_space=pltpu.SMEM),
            full((_D_IN, _HID)),
            full((_HID, _HEADS)),
            full((_HID, _HEADS)),
            full((1, _HID)),
            full((_HID, _HID)),
            full((_HID, _HEADS)),
            full((_HID, _HEADS)),
            full((1, _HID)),
        ],
        out_specs=pl.BlockSpec((1, _S, _HID), lambda b: (b, 0, 0)),
        out_shape=jax.ShapeDtypeStruct((_B, _S, _HID), jnp.float32),
        scratch_shapes=[
            pltpu.VMEM((2, _S, _S), jnp.int32),
            pltpu.SemaphoreType.DMA((2,)),
        ],
    )(sent_emb, adj_mask, sent_counts, W1, As1, Ad1, b1.reshape(1, _HID),
      W2, As2, Ad2, b2.reshape(1, _HID))
    return out


# speedup vs baseline: 1.0690x; 1.0690x over previous
import jax
import jax.numpy as jnp
from jax.experimental import pallas as pl
from jax.experimental.pallas import tpu as pltpu

_B, _S, _D_IN, _HID, _HEADS = 16, 512, 256, 256, 4
_DH = _HID // _HEADS
_NEG = -1e9
_L = 1.4426950408889634
_NEGL = _NEG * _L
_BUCKETS = (64, 128, 192, 256, 320, 384, 448, 512)


def _for_bucket(cnt, fn):
    lo = 0
    for iw in _BUCKETS:
        cond = (cnt <= iw) if lo == 0 else ((cnt > lo) & (cnt <= iw))
        pl.when(cond)(lambda iw=iw: fn(iw))
        lo = iw


def _body(x_ref, adj_ref, counts_ref, W1_ref, As1_ref, Ad1_ref, b1_ref,
          W2_ref, As2_ref, Ad2_ref, b2_ref, out_ref):
    b = pl.program_id(0)
    n = counts_ref[b]

    def arm(iw):
        adq = adj_ref[0, :iw, :iw].astype(jnp.float32)
        adqt = adq.T
        jj = jax.lax.broadcasted_iota(jnp.int32, (iw, iw), 0)
        ii = jax.lax.broadcasted_iota(jnp.int32, (iw, iw), 1)
        mb = jnp.where((adqt > 0.5) | (ii == jj), 0.0, _NEGL)
        icol = jax.lax.broadcasted_iota(jnp.int32, (1, iw), 1)

        def gat(xin, W_ref, As_ref, Ad_ref, b_ref):
            h = jnp.dot(xin, W_ref[...], preferred_element_type=jnp.float32)
            al_s = jnp.dot(h, As_ref[...], preferred_element_type=jnp.float32)
            al_d = jnp.dot(h, Ad_ref[...], preferred_element_type=jnp.float32)
            ones = jnp.ones((iw, 1), jnp.float32)
            al_sr = jnp.where(icol < n, al_s.T, _NEG)
            as1 = _L * al_sr
            as2 = (0.2 * _L) * al_sr
            amax = jnp.max(al_sr, axis=1, keepdims=True)
            outs = []
            for hd in range(_HEADS):
                ad = al_d[:, hd:hd + 1]
                m1 = ad + amax[hd:hd + 1, :]
                mx = jnp.maximum(m1, 0.2 * m1)
                ad1 = _L * (ad - mx)
                ad2 = _L * (0.2 * ad - mx)
                t = jnp.maximum(as1[hd:hd + 1, :] + ad1,
                                as2[hd:hd + 1, :] + ad2) + mb
                ex = jnp.exp2(t)
                hh = jnp.concatenate(
                    [h[:, hd * _DH:(hd + 1) * _DH], ones], axis=1)
                mm = jnp.dot(ex, hh, preferred_element_type=jnp.float32)
                rden = 1.0 / jnp.maximum(mm[:, _DH:_DH + 1], 1e-30)
                outs.append(mm[:, :_DH] * rden)
            return jnp.concatenate(outs, axis=1) + b_ref[...]

        x1 = gat(x_ref[0, :iw, :], W1_ref, As1_ref, Ad1_ref, b1_ref)
        x2 = gat(x1, W2_ref, As2_ref, Ad2_ref, b2_ref)
        rows = jax.lax.broadcasted_iota(jnp.int32, (iw, 1), 0)
        out_ref[0, :iw, :] = x2 * (rows < n).astype(jnp.float32)
        if iw < _S:
            out_ref[0, iw:, :] = jnp.zeros((_S - iw, _HID), jnp.float32)

    _for_bucket(n, arm)


def _head_mat(a):
    k = jnp.arange(_HID)
    sel = (k[:, None] // _DH) == jnp.arange(_HEADS)[None, :]
    return a.reshape(_HID)[:, None] * sel.astype(a.dtype)


def kernel(sent_emb, adj_mask, sent_counts, W1, a1_src, a1_dst, b1,
           W2, a2_src, a2_dst, b2):
    As1, Ad1 = _head_mat(a1_src), _head_mat(a1_dst)
    As2, Ad2 = _head_mat(a2_src), _head_mat(a2_dst)
    full = lambda shape: pl.BlockSpec(shape, lambda b: (0,) * len(shape))
    out = pl.pallas_call(
        _body,
        grid=(_B,),
        in_specs=[
            pl.BlockSpec((1, _S, _D_IN), lambda b: (b, 0, 0)),
            pl.BlockSpec((1, _S, _S), lambda b: (b, 0, 0)),
            pl.BlockSpec(memory_space=pltpu.SMEM),
            full((_D_IN, _HID)),
            full((_HID, _HEADS)),
            full((_HID, _HEADS)),
            full((1, _HID)),
            full((_HID, _HID)),
            full((_HID, _HEADS)),
            full((_HID, _HEADS)),
            full((1, _HID)),
        ],
        out_specs=pl.BlockSpec((1, _S, _HID), lambda b: (b, 0, 0)),
        out_shape=jax.ShapeDtypeStruct((_B, _S, _HID), jnp.float32),
    )(sent_emb, adj_mask, sent_counts, W1, As1, Ad1, b1.reshape(1, _HID),
      W2, As2, Ad2, b2.reshape(1, _HID))
    return out
